# baseline (device time: 66293 ns/iter reference)
import jax
import jax.numpy as jnp
from jax import lax
from jax.experimental import pallas as pl
from jax.experimental.pallas import tpu as pltpu

N_DEV = 16
W_SLOTS = 4

_PLANE = ((0, 0), (1, 0), (1, 1), (0, 1))
_COORDS = [(*_PLANE[p % 4], p // 4) for p in range(N_DEV)]


def _dist(a, b):
    (ax, ay, az), (bx, by, bz) = _COORDS[a], _COORDS[b]
    return abs(ax - bx) + abs(ay - by) + abs(az - bz)


_ORDER = [
    [m]
    + sorted(
        (j for j in range(N_DEV) if j != m),
        key=lambda j, m=m: (_dist(m, j), (j - m) % N_DEV),
    )
    for m in range(N_DEV)
]


def kernel(x, w_mat):
    k_total, k_shard = x.shape
    _, n = w_mat.shape
    m_out = k_total // N_DEV

    me_outer = lax.axis_index("i")
    order_row = jnp.asarray(_ORDER, jnp.int32)[me_outer]

    def body(order_ref, x_ref, w_hbm, out_ref,
             xbf_ref, recv_buf, w_buf, send_sems, recv_sems, w_sems):
        me = lax.axis_index("i")

        def load_w(j, slot):
            return pltpu.make_async_copy(
                w_hbm.at[pl.ds(j * m_out, m_out), :],
                w_buf.at[slot],
                w_sems.at[slot],
            )

        pending = [None] * W_SLOTS
        for k in range(W_SLOTS):
            pending[k] = load_w(order_ref[k], k)
            pending[k].start()

        xbf_ref[:, :] = x_ref[:, :].astype(jnp.bfloat16)

        for k in range(1, N_DEV):
            dst = order_ref[k]
            pltpu.make_async_remote_copy(
                src_ref=xbf_ref.at[pl.ds(dst * m_out, m_out), :],
                dst_ref=recv_buf.at[me],
                send_sem=send_sems.at[dst],
                recv_sem=recv_sems.at[me],
                device_id=(dst,),
                device_id_type=pl.DeviceIdType.MESH,
            ).start()

        for k in range(N_DEV):
            slot = k % W_SLOTS
            j = order_ref[k]
            if k == 0:
                a = xbf_ref[pl.ds(me * m_out, m_out), :]
            else:
                pltpu.make_async_remote_copy(
                    src_ref=xbf_ref.at[pl.ds(0, m_out), :],
                    dst_ref=recv_buf.at[j],
                    send_sem=send_sems.at[0],
                    recv_sem=recv_sems.at[j],
                    device_id=(me,),
                    device_id_type=pl.DeviceIdType.MESH,
                ).wait_recv()
                a = recv_buf[j]
            pending[slot].wait()
            acc = jnp.dot(a.astype(jnp.float32), w_buf[slot],
                          preferred_element_type=jnp.float32)
            if k == 0:
                out_ref[:, :] = acc
            else:
                out_ref[:, :] += acc
            if k + W_SLOTS < N_DEV:
                pending[slot] = load_w(order_ref[k + W_SLOTS], slot)
                pending[slot].start()

        for k in range(1, N_DEV):
            dst = order_ref[k]
            pltpu.make_async_remote_copy(
                src_ref=xbf_ref.at[pl.ds(dst * m_out, m_out), :],
                dst_ref=recv_buf.at[me],
                send_sem=send_sems.at[dst],
                recv_sem=recv_sems.at[me],
                device_id=(dst,),
                device_id_type=pl.DeviceIdType.MESH,
            ).wait_send()

    return pl.pallas_call(
        body,
        out_shape=jax.ShapeDtypeStruct((m_out, n), jnp.float32),
        in_specs=[
            pl.BlockSpec(memory_space=pltpu.SMEM),
            pl.BlockSpec(memory_space=pltpu.VMEM),
            pl.BlockSpec(memory_space=pl.ANY),
        ],
        out_specs=pl.BlockSpec(memory_space=pltpu.VMEM),
        scratch_shapes=[
            pltpu.VMEM((k_total, k_shard), jnp.bfloat16),
            pltpu.VMEM((N_DEV, m_out, k_shard), jnp.bfloat16),
            pltpu.VMEM((W_SLOTS, m_out, n), jnp.float32),
            pltpu.SemaphoreType.DMA((N_DEV,)),
            pltpu.SemaphoreType.DMA((N_DEV,)),
            pltpu.SemaphoreType.DMA((W_SLOTS,)),
        ],
        compiler_params=pltpu.CompilerParams(
            vmem_limit_bytes=100 * 1024 * 1024,
        ),
    )(order_row, x, w_mat)


# device time: 59321 ns/iter; 1.1175x vs baseline; 1.1175x over previous
import jax
import jax.numpy as jnp
from jax import lax
from jax.experimental import pallas as pl
from jax.experimental.pallas import tpu as pltpu

N_DEV = 16
W_SLOTS = 3

_PLANE = ((0, 0), (1, 0), (1, 1), (0, 1))
_COORDS = [(*_PLANE[p % 4], p // 4) for p in range(N_DEV)]


def _dist(a, b):
    (ax, ay, az), (bx, by, bz) = _COORDS[a], _COORDS[b]
    return abs(ax - bx) + abs(ay - by) + abs(az - bz)


_ORDER = [
    [m]
    + sorted(
        (j for j in range(N_DEV) if j != m),
        key=lambda j, m=m: (_dist(m, j), (j - m) % N_DEV),
    )
    for m in range(N_DEV)
]


def kernel(x, w_mat):
    k_total, k_shard = x.shape
    _, n = w_mat.shape
    m_out = k_total // N_DEV

    me_outer = lax.axis_index("i")
    order_row = jnp.asarray(_ORDER, jnp.int32)[me_outer]

    def body(order_ref, x_ref, w_hbm, out_ref,
             xbf_ref, recv_buf, w_buf, send_sems, recv_sems, w_sems):
        me = lax.axis_index("i")

        def load_w(j, slot):
            return pltpu.make_async_copy(
                w_hbm.at[pl.ds(j * m_out, m_out), :],
                w_buf.at[slot],
                w_sems.at[slot],
            )

        pending = [None] * W_SLOTS
        for k in range(W_SLOTS):
            pending[k] = load_w(order_ref[k], k)
            pending[k].start()

        xbf_ref[:, :] = x_ref[:, :].astype(jnp.bfloat16)

        for k in range(1, N_DEV):
            dst = order_ref[k]
            pltpu.make_async_remote_copy(
                src_ref=xbf_ref.at[pl.ds(dst * m_out, m_out), :],
                dst_ref=recv_buf.at[me],
                send_sem=send_sems.at[dst],
                recv_sem=recv_sems.at[me],
                device_id=(dst,),
                device_id_type=pl.DeviceIdType.MESH,
            ).start()

        for k in range(N_DEV):
            slot = k % W_SLOTS
            j = order_ref[k]
            if k == 0:
                a = xbf_ref[pl.ds(me * m_out, m_out), :]
            else:
                pltpu.make_async_remote_copy(
                    src_ref=xbf_ref.at[pl.ds(0, m_out), :],
                    dst_ref=recv_buf.at[j],
                    send_sem=send_sems.at[0],
                    recv_sem=recv_sems.at[j],
                    device_id=(me,),
                    device_id_type=pl.DeviceIdType.MESH,
                ).wait_recv()
                a = recv_buf[j]
            pending[slot].wait()
            acc = jnp.dot(a.astype(jnp.float32), w_buf[slot],
                          preferred_element_type=jnp.float32)
            if k == 0:
                out_ref[:, :] = acc
            else:
                out_ref[:, :] += acc
            if k + W_SLOTS < N_DEV:
                pending[slot] = load_w(order_ref[k + W_SLOTS], slot)
                pending[slot].start()

        for k in range(1, N_DEV):
            dst = order_ref[k]
            pltpu.make_async_remote_copy(
                src_ref=xbf_ref.at[pl.ds(dst * m_out, m_out), :],
                dst_ref=recv_buf.at[me],
                send_sem=send_sems.at[dst],
                recv_sem=recv_sems.at[me],
                device_id=(dst,),
                device_id_type=pl.DeviceIdType.MESH,
            ).wait_send()

    return pl.pallas_call(
        body,
        out_shape=jax.ShapeDtypeStruct((m_out, n), jnp.float32),
        in_specs=[
            pl.BlockSpec(memory_space=pltpu.SMEM),
            pl.BlockSpec(memory_space=pltpu.VMEM),
            pl.BlockSpec(memory_space=pl.ANY),
        ],
        out_specs=pl.BlockSpec(memory_space=pltpu.VMEM),
        scratch_shapes=[
            pltpu.VMEM((k_total, k_shard), jnp.bfloat16),
            pltpu.VMEM((N_DEV, m_out, k_shard), jnp.bfloat16),
            pltpu.VMEM((W_SLOTS, m_out, n), jnp.float32),
            pltpu.SemaphoreType.DMA((N_DEV,)),
            pltpu.SemaphoreType.DMA((N_DEV,)),
            pltpu.SemaphoreType.DMA((W_SLOTS,)),
        ],
    )(order_row, x, w_mat)


# device time: 51734 ns/iter; 1.2814x vs baseline; 1.1467x over previous
import jax
import jax.numpy as jnp
from jax import lax
from jax.experimental import pallas as pl
from jax.experimental.pallas import tpu as pltpu

N_DEV = 16
W_SLOTS = 3

_PLANE = ((0, 0), (1, 0), (1, 1), (0, 1))
_COORDS = [(*_PLANE[p % 4], p // 4) for p in range(N_DEV)]


def _dist(a, b):
    (ax, ay, az), (bx, by, bz) = _COORDS[a], _COORDS[b]
    return abs(ax - bx) + abs(ay - by) + abs(az - bz)


_ORDER = [
    [m]
    + sorted(
        (j for j in range(N_DEV) if j != m),
        key=lambda j, m=m: (_dist(m, j), (j - m) % N_DEV),
    )
    for m in range(N_DEV)
]


def kernel(x, w_mat):
    k_total, k_shard = x.shape
    _, n = w_mat.shape
    m_out = k_total // N_DEV

    me_outer = lax.axis_index("i")
    order_row = jnp.asarray(_ORDER, jnp.int32)[me_outer]

    def body(order_ref, x_ref, w_hbm, out_ref,
             xbf_ref, recv_buf, w_buf, send_sems, recv_sems, w_sems):
        me = lax.axis_index("i")

        def load_w(j, slot):
            return pltpu.make_async_copy(
                w_hbm.at[pl.ds(j * m_out, m_out), :],
                w_buf.at[slot],
                w_sems.at[slot],
            )

        pending = [None] * W_SLOTS
        for k in range(W_SLOTS):
            pending[k] = load_w(order_ref[k], k)
            pending[k].start()

        xbf_ref[:, :] = x_ref[:, :].astype(jnp.bfloat16)

        for k in range(1, N_DEV):
            dst = order_ref[k]
            pltpu.make_async_remote_copy(
                src_ref=xbf_ref.at[pl.ds(dst * m_out, m_out), :],
                dst_ref=recv_buf.at[me],
                send_sem=send_sems.at[dst],
                recv_sem=recv_sems.at[me],
                device_id=(dst,),
                device_id_type=pl.DeviceIdType.MESH,
            ).start()

        for k in range(N_DEV):
            slot = k % W_SLOTS
            j = order_ref[k]
            if k == 0:
                a = xbf_ref[pl.ds(me * m_out, m_out), :]
            else:
                pltpu.make_async_remote_copy(
                    src_ref=xbf_ref.at[pl.ds(0, m_out), :],
                    dst_ref=recv_buf.at[j],
                    send_sem=send_sems.at[0],
                    recv_sem=recv_sems.at[j],
                    device_id=(me,),
                    device_id_type=pl.DeviceIdType.MESH,
                ).wait_recv()
                a = recv_buf[j]
            pending[slot].wait()
            acc = jnp.dot(a.astype(jnp.float32), w_buf[slot],
                          preferred_element_type=jnp.float32)
            if k == 0:
                out_ref[:, :] = acc
            else:
                out_ref[:, :] += acc
            if k + W_SLOTS < N_DEV:
                pending[slot] = load_w(order_ref[k + W_SLOTS], slot)
                pending[slot].start()

        for k in range(1, N_DEV):
            dst = order_ref[k]
            pltpu.make_async_remote_copy(
                src_ref=xbf_ref.at[pl.ds(dst * m_out, m_out), :],
                dst_ref=recv_buf.at[me],
                send_sem=send_sems.at[dst],
                recv_sem=recv_sems.at[me],
                device_id=(dst,),
                device_id_type=pl.DeviceIdType.MESH,
            ).wait_send()

    return pl.pallas_call(
        body,
        out_shape=jax.ShapeDtypeStruct((m_out, n), jnp.float32),
        in_specs=[
            pl.BlockSpec(memory_space=pltpu.SMEM),
            pl.BlockSpec(memory_space=pltpu.VMEM),
            pl.BlockSpec(memory_space=pl.ANY),
        ],
        out_specs=pl.BlockSpec(memory_space=pltpu.VMEM),
        scratch_shapes=[
            pltpu.VMEM((k_total, k_shard), jnp.bfloat16),
            pltpu.VMEM((N_DEV, m_out, k_shard), jnp.bfloat16),
            pltpu.VMEM((W_SLOTS, m_out, n), jnp.float32),
            pltpu.SemaphoreType.DMA((N_DEV,)),
            pltpu.SemaphoreType.DMA((N_DEV,)),
            pltpu.SemaphoreType.DMA((W_SLOTS,)),
        ],
        compiler_params=pltpu.CompilerParams(
            skip_device_barrier=True,
        ),
    )(order_row, x, w_mat)
